# Initial kernel scaffold; baseline (speedup 1.0000x reference)
import jax
import jax.numpy as jnp
from jax.experimental import pallas as pl


def kernel(x, edge_index, batch, W11, b11, W12, b12, g1, be1, W21, b21, W22, b22, g2, be2, W31, b31, W32, b32, g3, be3, Wl1, bl1, Wl2, bl2):
    raise NotImplementedError("write your pallas kernel here")


# speedup vs baseline: 3.2956x; 3.2956x over previous
import functools

import jax
import jax.numpy as jnp
from jax import lax
from jax.experimental import pallas as pl
from jax.experimental.pallas import tpu as pltpu
from jax.experimental.pallas import tpu_sc as plsc

NC = 2
NS = 16
CH = 128
NGRAPH = 64


@functools.partial(jax.jit, static_argnums=(3, 4))
def _sc_agg(h, srcp, dstp, n_nodes, nchunk):
    dw = h.shape[1]
    mesh = plsc.VectorSubcoreMesh(core_axis_name="c", subcore_axis_name="s",
                                  num_cores=NC, num_subcores=NS)
    rpt = (-(-(n_nodes // 8) // NS)) * 8
    rpt_last = n_nodes - (NS - 1) * rpt

    @functools.partial(
        pl.kernel,
        out_type=[jax.ShapeDtypeStruct((n_nodes, dw), jnp.float32),
                  jax.ShapeDtypeStruct((n_nodes, dw), jnp.float32)],
        mesh=mesh,
        scratch_types=[
            pltpu.VMEM_SHARED((n_nodes + 8, dw), jnp.float32),
            pltpu.VMEM((1, CH), jnp.int32),
            pltpu.VMEM((1, CH), jnp.int32),
            pltpu.VMEM((CH, dw), jnp.float32),
        ],
    )
    def agg(h_hbm, src_hbm, dst_hbm, out0, out1, acc, srcv, dstv, rows):
        c = lax.axis_index("c")
        s = lax.axis_index("s")
        wid = c * NS + s
        @pl.when(s < NS - 1)
        def _():
            pltpu.sync_copy(h_hbm.at[pl.ds(s * rpt, rpt)],
                            acc.at[pl.ds(s * rpt, rpt)])

        @pl.when(s == NS - 1)
        def _():
            pltpu.sync_copy(h_hbm.at[pl.ds((NS - 1) * rpt, rpt_last)],
                            acc.at[pl.ds((NS - 1) * rpt, rpt_last)])

        plsc.subcore_barrier()
        base = wid * (nchunk * CH)

        @pl.loop(0, nchunk)
        def _(i):
            off = base + i * CH
            pltpu.sync_copy(src_hbm.at[pl.ds(off, CH)], srcv.at[0])
            pltpu.sync_copy(dst_hbm.at[pl.ds(off, CH)], dstv.at[0])
            pltpu.sync_copy(h_hbm.at[srcv.at[0]], rows)
            pltpu.sync_copy(rows, acc.at[dstv.at[0]], add=True)

        plsc.subcore_barrier()

        @pl.when(jnp.logical_and(s == 0, c == 0))
        def _():
            pltpu.sync_copy(acc.at[pl.ds(0, n_nodes)], out0)

        @pl.when(jnp.logical_and(s == 0, c == 1))
        def _():
            pltpu.sync_copy(acc.at[pl.ds(0, n_nodes)], out1)

    return agg(h, srcp, dstp)


def _mlp_body(a0_ref, a1_ref, hp_ref, w1_ref, b1_ref, w2_ref, b2_ref,
              g_ref, be_ref, hpre_ref, stats_ref, acc_ref, *, n_nodes):
    i = pl.program_id(0)
    hin = a0_ref[...] + a1_ref[...] - hp_ref[...]
    t = jnp.dot(hin, w1_ref[...], preferred_element_type=jnp.float32)
    t = jax.nn.gelu(t + b1_ref[...])
    hpre = jnp.dot(t, w2_ref[...], preferred_element_type=jnp.float32)
    hpre = hpre + b2_ref[...]
    hpre_ref[...] = hpre
    ps = jnp.sum(hpre, axis=0)
    pq = jnp.sum(hpre * hpre, axis=0)

    @pl.when(i == 0)
    def _():
        acc_ref[...] = jnp.zeros_like(acc_ref)

    acc_ref[0] += ps
    acc_ref[1] += pq

    @pl.when(i == pl.num_programs(0) - 1)
    def _():
        mu = acc_ref[0] / n_nodes
        var = acc_ref[1] / n_nodes - mu * mu
        scale = g_ref[0] * lax.rsqrt(var + 1e-5)
        stats_ref[0] = scale
        stats_ref[1] = be_ref[0] - mu * scale
        stats_ref[2:] = jnp.zeros_like(stats_ref[2:])


def _tc_mlp(a0, a1, hprev, w1, b1, w2, b2, g, be, br):
    n_nodes, din = hprev.shape
    k = w1.shape[1]
    grid = (n_nodes // br,)
    row = lambda i: (i, 0)
    fix = lambda i: (0, 0)
    return pl.pallas_call(
        functools.partial(_mlp_body, n_nodes=n_nodes),
        grid=grid,
        in_specs=[
            pl.BlockSpec((br, din), row),
            pl.BlockSpec((br, din), row),
            pl.BlockSpec((br, din), row),
            pl.BlockSpec((din, k), fix),
            pl.BlockSpec((1, k), fix),
            pl.BlockSpec((k, k), fix),
            pl.BlockSpec((1, k), fix),
            pl.BlockSpec((1, k), fix),
            pl.BlockSpec((1, k), fix),
        ],
        out_specs=[
            pl.BlockSpec((br, k), row),
            pl.BlockSpec((8, k), fix),
        ],
        out_shape=[
            jax.ShapeDtypeStruct((n_nodes, k), jnp.float32),
            jax.ShapeDtypeStruct((8, k), jnp.float32),
        ],
        scratch_shapes=[pltpu.VMEM((8, k), jnp.float32)],
    )(a0, a1, hprev, w1, b1, w2, b2, g, be)


def _bn_body(hpre_ref, stats_ref, batch_ref, h_ref, p_ref):
    i = pl.program_id(0)
    hb = hpre_ref[...] * stats_ref[0] + stats_ref[1]
    hb = jax.nn.gelu(hb)
    h_ref[...] = hb
    b = batch_ref[0, 0]
    oh = (b[:, None] == lax.broadcasted_iota(jnp.int32, (b.shape[0], NGRAPH), 1))
    oh = oh.astype(jnp.float32)
    pp = lax.dot_general(oh, hb, (((0,), (0,)), ((), ())),
                         preferred_element_type=jnp.float32)

    @pl.when(i == 0)
    def _():
        p_ref[...] = pp

    @pl.when(i > 0)
    def _():
        p_ref[...] += pp


def _tc_bn(hpre, stats, batch3, br):
    n_nodes, k = hpre.shape
    grid = (n_nodes // br,)
    return pl.pallas_call(
        _bn_body,
        grid=grid,
        in_specs=[
            pl.BlockSpec((br, k), lambda i: (i, 0)),
            pl.BlockSpec((8, k), lambda i: (0, 0)),
            pl.BlockSpec((1, 1, br), lambda i: (i, 0, 0)),
        ],
        out_specs=[
            pl.BlockSpec((br, k), lambda i: (i, 0)),
            pl.BlockSpec((NGRAPH, k), lambda i: (0, 0)),
        ],
        out_shape=[
            jax.ShapeDtypeStruct((n_nodes, k), jnp.float32),
            jax.ShapeDtypeStruct((NGRAPH, k), jnp.float32),
        ],
    )(hpre, stats, batch3)


def _readout_body(p1_ref, p2_ref, p3_ref, wl1_ref, bl1_ref, wl2_ref, bl2_ref,
                  out_ref):
    pc = jnp.concatenate([p1_ref[...], p2_ref[...], p3_ref[...]], axis=1)
    hh = jnp.dot(pc, wl1_ref[...], preferred_element_type=jnp.float32)
    hh = jnp.maximum(hh + bl1_ref[...], 0.0)
    out = jnp.dot(hh, wl2_ref[...], preferred_element_type=jnp.float32)
    out_ref[...] = out + bl2_ref[...]


def _tc_readout(p1, p2, p3, wl1, bl1, wl2, bl2):
    c = wl2.shape[1]
    return pl.pallas_call(
        _readout_body,
        out_shape=jax.ShapeDtypeStruct((NGRAPH, c), jnp.float32),
    )(p1, p2, p3, wl1, bl1, wl2, bl2)


def kernel(x, edge_index, batch, W11, b11, W12, b12, g1, be1,
           W21, b21, W22, b22, g2, be2,
           W31, b31, W32, b32, g3, be3,
           Wl1, bl1, Wl2, bl2):
    n, d = x.shape
    e = edge_index.shape[1]
    nw = NC * NS
    nchunk = -(-e // (nw * CH))
    epad = nw * nchunk * CH
    src = edge_index[0]
    dst = edge_index[1]
    if epad > e:
        pad = epad - e
        src = jnp.concatenate([src, jnp.zeros((pad,), jnp.int32)])
        dst = jnp.concatenate([dst, jnp.full((pad,), n, jnp.int32)])

    br = 1000
    batch3 = batch.reshape(n // br, 1, br)
    r2 = lambda v: v.reshape(1, -1)

    a0, a1 = _sc_agg(x, src, dst, n, nchunk)
    hpre1, st1 = _tc_mlp(a0, a1, x, W11, r2(b11), W12, r2(b12),
                         r2(g1), r2(be1), br)
    h1, p1 = _tc_bn(hpre1, st1, batch3, br)

    a0, a1 = _sc_agg(h1, src, dst, n, nchunk)
    hpre2, st2 = _tc_mlp(a0, a1, h1, W21, r2(b21), W22, r2(b22),
                         r2(g2), r2(be2), br)
    h2, p2 = _tc_bn(hpre2, st2, batch3, br)

    a0, a1 = _sc_agg(h2, src, dst, n, nchunk)
    hpre3, st3 = _tc_mlp(a0, a1, h2, W31, r2(b31), W32, r2(b32),
                         r2(g3), r2(be3), br)
    h3, p3 = _tc_bn(hpre3, st3, batch3, br)

    return _tc_readout(p1, p2, p3, Wl1, r2(bl1), Wl2, r2(bl2))
